# baseline (speedup 1.0000x reference)
import functools

import jax
import jax.numpy as jnp
from jax.experimental import pallas as pl
from jax.experimental.pallas import tpu as pltpu

HIDDEN = 960
VOCAB_PAD = 1024
EPS = 1e-12


def _body(ids_ref, mask_ref, pe_ref, w_ref, b_ref, me_ref, tbl_ref, g_ref,
          bt_ref, out_ref, *, block_rows):
    x = jax.lax.dot_general(
        pe_ref[...], w_ref[...], (((1,), (1,)), ((), ())),
        preferred_element_type=jnp.float32)
    x = x + b_ref[...]
    m = mask_ref[0] > 0
    x = jnp.where(m, me_ref[...], x)
    ids = ids_ref[0]
    onehot = (ids == jax.lax.broadcasted_iota(
        jnp.int32, (block_rows, VOCAB_PAD), 1)).astype(jnp.float32)
    x = x + jnp.dot(onehot, tbl_ref[...], preferred_element_type=jnp.float32)
    mean = jnp.mean(x, axis=1, keepdims=True)
    xc = x - mean
    var = jnp.mean(xc * xc, axis=1, keepdims=True)
    out_ref[...] = xc * jax.lax.rsqrt(var + EPS) * g_ref[...] + bt_ref[...]


@jax.jit
def kernel(protein_embeddings, contig_ids, mlm_mask, W, b, mask_embed,
           contig_table, ln_gamma, ln_beta):
    bsz, seq, hidden = protein_embeddings.shape
    n = bsz * seq

    block_rows = 1200
    if n % block_rows != 0:
        block_rows = 8
    n_pad = ((n + block_rows - 1) // block_rows) * block_rows
    nblk = n_pad // block_rows

    pe = protein_embeddings.reshape(n, hidden)
    ids = contig_ids.reshape(n).astype(jnp.int32)
    mask = mlm_mask.reshape(n).astype(jnp.int32)
    if n_pad != n:
        pe = jnp.pad(pe, ((0, n_pad - n), (0, 0)))
        ids = jnp.pad(ids, (0, n_pad - n))
        mask = jnp.pad(mask, (0, n_pad - n))
    ids = ids.reshape(nblk, block_rows, 1)
    mask = mask.reshape(nblk, block_rows, 1)

    tbl = jnp.zeros((VOCAB_PAD, hidden), jnp.float32).at[:contig_table.shape[0]].set(contig_table)

    row = lambda v: v.reshape(1, hidden)

    out = pl.pallas_call(
        functools.partial(_body, block_rows=block_rows),
        grid=(nblk,),
        in_specs=[
            pl.BlockSpec((1, block_rows, 1), lambda i: (i, 0, 0)),
            pl.BlockSpec((1, block_rows, 1), lambda i: (i, 0, 0)),
            pl.BlockSpec((block_rows, hidden), lambda i: (i, 0)),
            pl.BlockSpec((hidden, hidden), lambda i: (0, 0)),
            pl.BlockSpec((1, hidden), lambda i: (0, 0)),
            pl.BlockSpec((1, hidden), lambda i: (0, 0)),
            pl.BlockSpec((VOCAB_PAD, hidden), lambda i: (0, 0)),
            pl.BlockSpec((1, hidden), lambda i: (0, 0)),
            pl.BlockSpec((1, hidden), lambda i: (0, 0)),
        ],
        out_specs=pl.BlockSpec((block_rows, hidden), lambda i: (i, 0)),
        out_shape=jax.ShapeDtypeStruct((n_pad, hidden), jnp.float32),
        compiler_params=pltpu.CompilerParams(
            dimension_semantics=("arbitrary",),
        ),
    )(ids, mask, pe, W, row(b), row(mask_embed), tbl, row(ln_gamma),
      row(ln_beta))

    return out[:n].reshape(bsz, seq, hidden)


# speedup vs baseline: 1.0021x; 1.0021x over previous
import functools

import jax
import jax.numpy as jnp
from jax.experimental import pallas as pl
from jax.experimental.pallas import tpu as pltpu

HIDDEN = 960
VOCAB_PAD = 1024
EPS = 1e-12


def _body(ids_ref, mask_ref, pe_ref, w_ref, b_ref, me_ref, tbl_ref, g_ref,
          bt_ref, out_ref, *, block_rows):
    x = jax.lax.dot_general(
        pe_ref[...].astype(jnp.bfloat16), w_ref[...].astype(jnp.bfloat16),
        (((1,), (1,)), ((), ())),
        preferred_element_type=jnp.float32)
    x = x + b_ref[...]
    m = mask_ref[0] > 0
    x = jnp.where(m, me_ref[...], x)
    ids = ids_ref[0]
    onehot = (ids == jax.lax.broadcasted_iota(
        jnp.int32, (block_rows, VOCAB_PAD), 1)).astype(jnp.bfloat16)
    x = x + jnp.dot(onehot, tbl_ref[...].astype(jnp.bfloat16),
                    preferred_element_type=jnp.float32)
    mean = jnp.mean(x, axis=1, keepdims=True)
    xc = x - mean
    var = jnp.mean(xc * xc, axis=1, keepdims=True)
    out_ref[...] = xc * jax.lax.rsqrt(var + EPS) * g_ref[...] + bt_ref[...]


@jax.jit
def kernel(protein_embeddings, contig_ids, mlm_mask, W, b, mask_embed,
           contig_table, ln_gamma, ln_beta):
    bsz, seq, hidden = protein_embeddings.shape
    n = bsz * seq

    block_rows = 1200
    if n % block_rows != 0:
        block_rows = 8
    n_pad = ((n + block_rows - 1) // block_rows) * block_rows
    nblk = n_pad // block_rows

    pe = protein_embeddings.reshape(n, hidden)
    ids = contig_ids.reshape(n).astype(jnp.int32)
    mask = mlm_mask.reshape(n).astype(jnp.int32)
    if n_pad != n:
        pe = jnp.pad(pe, ((0, n_pad - n), (0, 0)))
        ids = jnp.pad(ids, (0, n_pad - n))
        mask = jnp.pad(mask, (0, n_pad - n))
    ids = ids.reshape(nblk, block_rows, 1)
    mask = mask.reshape(nblk, block_rows, 1)

    tbl = jnp.zeros((VOCAB_PAD, hidden), jnp.float32).at[:contig_table.shape[0]].set(contig_table)

    row = lambda v: v.reshape(1, hidden)

    out = pl.pallas_call(
        functools.partial(_body, block_rows=block_rows),
        grid=(nblk,),
        in_specs=[
            pl.BlockSpec((1, block_rows, 1), lambda i: (i, 0, 0)),
            pl.BlockSpec((1, block_rows, 1), lambda i: (i, 0, 0)),
            pl.BlockSpec((block_rows, hidden), lambda i: (i, 0)),
            pl.BlockSpec((hidden, hidden), lambda i: (0, 0)),
            pl.BlockSpec((1, hidden), lambda i: (0, 0)),
            pl.BlockSpec((1, hidden), lambda i: (0, 0)),
            pl.BlockSpec((VOCAB_PAD, hidden), lambda i: (0, 0)),
            pl.BlockSpec((1, hidden), lambda i: (0, 0)),
            pl.BlockSpec((1, hidden), lambda i: (0, 0)),
        ],
        out_specs=pl.BlockSpec((block_rows, hidden), lambda i: (i, 0)),
        out_shape=jax.ShapeDtypeStruct((n_pad, hidden), jnp.float32),
        compiler_params=pltpu.CompilerParams(
            dimension_semantics=("arbitrary",),
        ),
    )(ids, mask, pe, W, row(b), row(mask_embed), tbl, row(ln_gamma),
      row(ln_beta))

    return out[:n].reshape(bsz, seq, hidden)


# speedup vs baseline: 1.0021x; 1.0000x over previous
import functools

import jax
import jax.numpy as jnp
from jax.experimental import pallas as pl
from jax.experimental.pallas import tpu as pltpu

EPS = 1e-12
MASK_BIT = 1 << 14


def _body(comb_ref, pe_ref, w_ref, b_ref, me_ref, tbl_ref, g_ref,
          bt_ref, out_ref, *, block_rows, vocab):
    comb = comb_ref[0]
    ids_row = comb & (MASK_BIT - 1)
    m_row = (comb >> 14).astype(jnp.float32)

    x = jax.lax.dot_general(
        pe_ref[...].astype(jnp.bfloat16), w_ref[...].astype(jnp.bfloat16),
        (((1,), (1,)), ((), ())),
        preferred_element_type=jnp.float32)
    x = x + b_ref[...]

    ones11 = jnp.ones((1, 1), jnp.float32)
    m_col = jax.lax.dot_general(
        m_row, ones11, (((0,), (0,)), ((), ())),
        preferred_element_type=jnp.float32)
    x = jnp.where(m_col > 0.5, me_ref[...], x)

    oh_t = (jax.lax.broadcasted_iota(jnp.int32, (vocab, block_rows), 0)
            == ids_row).astype(jnp.bfloat16)
    x = x + jax.lax.dot_general(
        oh_t, tbl_ref[...].astype(jnp.bfloat16),
        (((0,), (0,)), ((), ())),
        preferred_element_type=jnp.float32)

    mean = jnp.mean(x, axis=1, keepdims=True)
    xc = x - mean
    var = jnp.mean(xc * xc, axis=1, keepdims=True)
    out_ref[...] = xc * jax.lax.rsqrt(var + EPS) * g_ref[...] + bt_ref[...]


@jax.jit
def kernel(protein_embeddings, contig_ids, mlm_mask, W, b, mask_embed,
           contig_table, ln_gamma, ln_beta):
    bsz, seq, hidden = protein_embeddings.shape
    vocab = contig_table.shape[0]
    n = bsz * seq

    block_rows = 1200
    if n % block_rows != 0:
        block_rows = 8
    n_pad = ((n + block_rows - 1) // block_rows) * block_rows
    nblk = n_pad // block_rows

    pe = protein_embeddings.reshape(n, hidden)
    comb = (contig_ids.astype(jnp.int32)
            + mlm_mask.astype(jnp.int32) * MASK_BIT).reshape(n)
    if n_pad != n:
        pe = jnp.pad(pe, ((0, n_pad - n), (0, 0)))
        comb = jnp.pad(comb, (0, n_pad - n))
    comb = comb.reshape(nblk, 1, block_rows)

    row = lambda v: v.reshape(1, hidden)

    out = pl.pallas_call(
        functools.partial(_body, block_rows=block_rows, vocab=vocab),
        grid=(nblk,),
        in_specs=[
            pl.BlockSpec((1, 1, block_rows), lambda i: (i, 0, 0)),
            pl.BlockSpec((block_rows, hidden), lambda i: (i, 0)),
            pl.BlockSpec((hidden, hidden), lambda i: (0, 0)),
            pl.BlockSpec((1, hidden), lambda i: (0, 0)),
            pl.BlockSpec((1, hidden), lambda i: (0, 0)),
            pl.BlockSpec((vocab, hidden), lambda i: (0, 0)),
            pl.BlockSpec((1, hidden), lambda i: (0, 0)),
            pl.BlockSpec((1, hidden), lambda i: (0, 0)),
        ],
        out_specs=pl.BlockSpec((block_rows, hidden), lambda i: (i, 0)),
        out_shape=jax.ShapeDtypeStruct((n_pad, hidden), jnp.float32),
        compiler_params=pltpu.CompilerParams(
            dimension_semantics=("arbitrary",),
        ),
    )(comb, pe, W, row(b), row(mask_embed), contig_table, row(ln_gamma),
      row(ln_beta))

    return out[:n].reshape(bsz, seq, hidden)


# speedup vs baseline: 2.7905x; 2.7846x over previous
import functools

import jax
import jax.numpy as jnp
from jax.experimental import pallas as pl
from jax.experimental.pallas import tpu as pltpu

EPS = 1e-12
MASK_BIT = 1 << 14


def _body(comb_ref, pe_ref, w_ref, b_ref, me_ref, tbl_ref, g_ref,
          bt_ref, out_ref, *, block_rows, vocab):
    ji = pl.program_id(0)
    bi = pl.program_id(1)
    bsz, seq = comb_ref.shape

    comb_all = comb_ref[...]
    rowsel = jax.lax.broadcasted_iota(jnp.int32, (bsz, 1), 0) == bi
    comb_row = jnp.sum(jnp.where(rowsel, comb_all, 0), axis=0,
                       keepdims=True).astype(jnp.float32)

    comb_win = comb_row[:, 0:block_rows]
    for k in range(1, seq // block_rows):
        comb_win = jnp.where(
            ji == k, comb_row[:, k * block_rows:(k + 1) * block_rows],
            comb_win)

    ones11 = jnp.ones((1, 1), jnp.float32)
    comb_col = jax.lax.dot_general(
        comb_win, ones11, (((0,), (0,)), ((), ())),
        precision=jax.lax.Precision.HIGHEST,
        preferred_element_type=jnp.float32)
    m_col = jnp.floor(comb_col * (1.0 / MASK_BIT))
    ids_col = comb_col - m_col * MASK_BIT

    x = jax.lax.dot_general(
        pe_ref[0].astype(jnp.bfloat16), w_ref[...].astype(jnp.bfloat16),
        (((1,), (1,)), ((), ())),
        preferred_element_type=jnp.float32)
    x = x + b_ref[...]
    x = jnp.where(m_col > 0.5, me_ref[...], x)

    oh = (ids_col.astype(jnp.int32) == jax.lax.broadcasted_iota(
        jnp.int32, (block_rows, vocab), 1)).astype(jnp.bfloat16)
    x = x + jax.lax.dot_general(
        oh, tbl_ref[...].astype(jnp.bfloat16),
        (((1,), (0,)), ((), ())),
        preferred_element_type=jnp.float32)

    mean = jnp.mean(x, axis=1, keepdims=True)
    xc = x - mean
    var = jnp.mean(xc * xc, axis=1, keepdims=True)
    out_ref[0] = xc * jax.lax.rsqrt(var + EPS) * g_ref[...] + bt_ref[...]


@jax.jit
def kernel(protein_embeddings, contig_ids, mlm_mask, W, b, mask_embed,
           contig_table, ln_gamma, ln_beta):
    bsz, seq, hidden = protein_embeddings.shape
    vocab = contig_table.shape[0]

    block_rows = seq
    for cand in range(min(1200, seq), 7, -1):
        if seq % cand == 0:
            block_rows = cand
            break
    nblk = seq // block_rows

    comb = (contig_ids.astype(jnp.int32)
            + mlm_mask.astype(jnp.int32) * MASK_BIT)

    row = lambda v: v.reshape(1, hidden)

    out = pl.pallas_call(
        functools.partial(_body, block_rows=block_rows, vocab=vocab),
        grid=(nblk, bsz),
        in_specs=[
            pl.BlockSpec((bsz, seq), lambda j, i: (0, 0)),
            pl.BlockSpec((1, block_rows, hidden), lambda j, i: (i, j, 0)),
            pl.BlockSpec((hidden, hidden), lambda j, i: (0, 0)),
            pl.BlockSpec((1, hidden), lambda j, i: (0, 0)),
            pl.BlockSpec((1, hidden), lambda j, i: (0, 0)),
            pl.BlockSpec((vocab, hidden), lambda j, i: (0, 0)),
            pl.BlockSpec((1, hidden), lambda j, i: (0, 0)),
            pl.BlockSpec((1, hidden), lambda j, i: (0, 0)),
        ],
        out_specs=pl.BlockSpec((1, block_rows, hidden), lambda j, i: (i, j, 0)),
        out_shape=jax.ShapeDtypeStruct((bsz, seq, hidden), jnp.float32),
        compiler_params=pltpu.CompilerParams(
            dimension_semantics=("arbitrary", "arbitrary"),
        ),
    )(comb, protein_embeddings, W, row(b), row(mask_embed), contig_table,
      row(ln_gamma), row(ln_beta))

    return out


# speedup vs baseline: 2.8021x; 1.0042x over previous
import functools

import jax
import jax.numpy as jnp
from jax.experimental import pallas as pl
from jax.experimental.pallas import tpu as pltpu

EPS = 1e-12
MASK_BIT = 1 << 14


def _body(comb_ref, pe_ref, w_ref, b_ref, me_ref, tbl_ref, g_ref,
          bt_ref, out_ref, *, block_rows, vocab):
    ji = pl.program_id(0)
    bi = pl.program_id(1)
    bsz, seq = comb_ref.shape

    comb_all = comb_ref[...]
    rowsel = jax.lax.broadcasted_iota(jnp.int32, (bsz, 1), 0) == bi
    comb_row = jnp.sum(jnp.where(rowsel, comb_all, 0), axis=0,
                       keepdims=True).astype(jnp.float32)

    comb_win = comb_row[:, 0:block_rows]
    for k in range(1, seq // block_rows):
        comb_win = jnp.where(
            ji == k, comb_row[:, k * block_rows:(k + 1) * block_rows],
            comb_win)

    ones11 = jnp.ones((1, 1), jnp.float32)
    comb_col = jax.lax.dot_general(
        comb_win, ones11, (((0,), (0,)), ((), ())),
        precision=jax.lax.Precision.HIGHEST,
        preferred_element_type=jnp.float32)
    m_col = jnp.floor(comb_col * (1.0 / MASK_BIT))
    ids_col = comb_col - m_col * MASK_BIT

    x = jax.lax.dot_general(
        pe_ref[0].astype(jnp.bfloat16), w_ref[...].astype(jnp.bfloat16),
        (((1,), (1,)), ((), ())),
        preferred_element_type=jnp.float32)
    x = x + b_ref[...]
    x = jnp.where(m_col > 0.5, me_ref[...], x)

    oh = (ids_col.astype(jnp.int32) == jax.lax.broadcasted_iota(
        jnp.int32, (block_rows, vocab), 1)).astype(jnp.bfloat16)
    x = x + jax.lax.dot_general(
        oh, tbl_ref[...].astype(jnp.bfloat16),
        (((1,), (0,)), ((), ())),
        preferred_element_type=jnp.float32)

    mean = jnp.mean(x, axis=1, keepdims=True)
    xc = x - mean
    var = jnp.mean(xc * xc, axis=1, keepdims=True)
    out_ref[0] = xc * jax.lax.rsqrt(var + EPS) * g_ref[...] + bt_ref[...]


@jax.jit
def kernel(protein_embeddings, contig_ids, mlm_mask, W, b, mask_embed,
           contig_table, ln_gamma, ln_beta):
    bsz, seq, hidden = protein_embeddings.shape
    vocab = contig_table.shape[0]

    block_rows = seq
    for cand in range(min(2000, seq), 7, -1):
        if seq % cand == 0 and cand % 8 == 0:
            block_rows = cand
            break
    nblk = seq // block_rows

    comb = (contig_ids.astype(jnp.int32)
            + mlm_mask.astype(jnp.int32) * MASK_BIT)

    row = lambda v: v.reshape(1, hidden)

    out = pl.pallas_call(
        functools.partial(_body, block_rows=block_rows, vocab=vocab),
        grid=(nblk, bsz),
        in_specs=[
            pl.BlockSpec((bsz, seq), lambda j, i: (0, 0)),
            pl.BlockSpec((1, block_rows, hidden), lambda j, i: (i, j, 0)),
            pl.BlockSpec((hidden, hidden), lambda j, i: (0, 0)),
            pl.BlockSpec((1, hidden), lambda j, i: (0, 0)),
            pl.BlockSpec((1, hidden), lambda j, i: (0, 0)),
            pl.BlockSpec((vocab, hidden), lambda j, i: (0, 0)),
            pl.BlockSpec((1, hidden), lambda j, i: (0, 0)),
            pl.BlockSpec((1, hidden), lambda j, i: (0, 0)),
        ],
        out_specs=pl.BlockSpec((1, block_rows, hidden), lambda j, i: (i, j, 0)),
        out_shape=jax.ShapeDtypeStruct((bsz, seq, hidden), jnp.float32),
        compiler_params=pltpu.CompilerParams(
            dimension_semantics=("arbitrary", "arbitrary"),
        ),
    )(comb, protein_embeddings, W, row(b), row(mask_embed), contig_table,
      row(ln_gamma), row(ln_beta))

    return out
